# baseline (speedup 1.0000x reference)
import functools

import jax
import jax.numpy as jnp
import numpy as np
from jax import lax
from jax.experimental import pallas as pl
from jax.experimental.pallas import tpu as pltpu
from jax.experimental.pallas import tpu_sc as plsc

B = 16384
NUM_SPARSE = 26
VOCAB = 100000
EMB_DIM = 16
N_FEAT = NUM_SPARSE + 1

_NC, _NS = 2, 16
_NW = _NC * _NS
_TOTAL_ROWS = B * NUM_SPARSE
_ROWS_PER_W = _TOTAL_ROWS // _NW
_PACKED_V = 325000
_CHUNK = 256
_NCHUNKS = _ROWS_PER_W // _CHUNK
_IDXROWS = _ROWS_PER_W // 128
_OUT_PACKED = _TOTAL_ROWS // 8
_OPW = _ROWS_PER_W // 8


def _sc_gather_body(table_hbm, idx_hbm, pidx_hbm, out_hbm,
                    idx_v, pidx_v, gbuf, obuf, sem):
    i32 = jnp.int32
    wid = lax.axis_index("s") * _NC + lax.axis_index("c")
    pltpu.sync_copy(idx_hbm.at[wid], idx_v)
    pltpu.sync_copy(pidx_hbm.at[wid], pidx_v)

    iota16 = lax.iota(i32, 16)
    lane_hi = lax.shift_right_logical(iota16, 3)
    cbase = lax.shift_left(lax.bitwise_and(iota16, 7), 4)
    obase = wid * _OPW

    def chunk(c, carry):
        hs = [
            pltpu.async_copy(
                table_hbm.at[pidx_v.at[2 * c + h]],
                gbuf.at[pl.ds(h * 128, 128)],
                sem,
            )
            for h in range(2)
        ]
        for h in hs:
            h.wait()
        for g in range(16):
            ov = lax.shift_left(
                lax.bitwise_and(idx_v[2 * c + g // 8, pl.ds((g % 8) * 16, 16)], 7),
                4)
            growvec = g * 16 + iota16
            orowvec = (lax.rem(c, 4) * 32 + 2 * g) + lane_hi
            for i in range(16):
                val = plsc.load_gather(gbuf, [growvec, ov + i])
                plsc.store_scatter(obuf, [orowvec, cbase + i], val)

        @pl.when(lax.rem(c, 4) == 3)
        def _flush():
            start = pl.multiple_of(obase + (c // 4) * 128, 128)
            pltpu.sync_copy(obuf, out_hbm.at[pl.ds(start, 128)])

        return carry

    lax.fori_loop(0, _NCHUNKS, chunk, 0)


@functools.cache
def _sc_gather():
    return pl.kernel(
        _sc_gather_body,
        mesh=plsc.VectorSubcoreMesh(core_axis_name="c", subcore_axis_name="s"),
        out_type=jax.ShapeDtypeStruct((_OUT_PACKED, 128), jnp.float32),
        scratch_types=[
            pltpu.VMEM((_IDXROWS, 128), jnp.int32),
            pltpu.VMEM((_IDXROWS, 128), jnp.int32),
            pltpu.VMEM((_CHUNK, 128), jnp.float32),
            pltpu.VMEM((128, 128), jnp.float32),
            pltpu.SemaphoreType.DMA,
        ],
        compiler_params=pltpu.CompilerParams(
            use_tc_tiling_on_sc=True, needs_layout_passes=False),
    )


_BB = 512


def _tc_dlrm_body(num_ref, emb_ref, b0w, b0b, b1w, b1b, b2w, b2b,
                  w0a, w0z, t0b, t1w, t1b, t2w, t2b, t3w, t3b, t4w, t4b,
                  out_ref):
    f32 = jnp.float32
    bf16 = jnp.bfloat16
    x = num_ref[...]
    h = jax.nn.relu(jnp.dot(x, b0w[...], preferred_element_type=f32) + b0b[...])
    h = jax.nn.relu(jnp.dot(h.astype(bf16), b1w[...], preferred_element_type=f32)
                    + b1b[...])
    bot = jax.nn.relu(jnp.dot(h, b2w[...], preferred_element_type=f32) + b2b[...])

    emb = emb_ref[...].reshape(_BB, NUM_SPARSE, EMB_DIM)
    xcat = jnp.concatenate([bot.reshape(_BB, 1, EMB_DIM), emb], axis=1)
    z = lax.dot_general(xcat, xcat, (((2,), (2,)), ((0,), (0,))),
                        preferred_element_type=f32)
    zflat = z.reshape(_BB, N_FEAT * N_FEAT)

    t = jnp.dot(bot, w0a[...], preferred_element_type=f32)
    t = t + jnp.dot(zflat.astype(bf16), w0z[...], preferred_element_type=f32)
    t = jax.nn.relu(t + t0b[...])
    t = jax.nn.relu(jnp.dot(t.astype(bf16), t1w[...], preferred_element_type=f32)
                    + t1b[...])
    t = jax.nn.relu(jnp.dot(t.astype(bf16), t2w[...], preferred_element_type=f32)
                    + t2b[...])
    t = jax.nn.relu(jnp.dot(t.astype(bf16), t3w[...], preferred_element_type=f32)
                    + t3b[...])
    out_ref[...] = jnp.dot(t, t4w[...], preferred_element_type=f32) + t4b[...]


def _tc_dlrm(num, emb2d, b0w, b0b, b1w, b1b, b2w, b2b,
             w0a, w0z, t0b, t1w, t1b, t2w, t2b, t3w, t3b, t4w, t4b):
    grid = (B // _BB,)
    full = lambda a: pl.BlockSpec(a.shape, lambda i: (0,) * a.ndim)
    in_specs = [
        pl.BlockSpec((_BB, num.shape[1]), lambda i: (i, 0)),
        pl.BlockSpec((_BB, emb2d.shape[1]), lambda i: (i, 0)),
    ] + [full(a) for a in (b0w, b0b, b1w, b1b, b2w, b2b,
                           w0a, w0z, t0b, t1w, t1b, t2w, t2b, t3w, t3b,
                           t4w, t4b)]
    return pl.pallas_call(
        _tc_dlrm_body,
        grid=grid,
        in_specs=in_specs,
        out_specs=pl.BlockSpec((_BB, 1), lambda i: (i, 0)),
        out_shape=jax.ShapeDtypeStruct((B, 1), jnp.float32),
    )(num, emb2d, b0w, b0b, b1w, b1b, b2w, b2b,
      w0a, w0z, t0b, t1w, t1b, t2w, t2b, t3w, t3b, t4w, t4b)


_LI, _LJ = np.tril_indices(N_FEAT, -1)
_PAIR_POS = np.asarray(_LI * N_FEAT + _LJ, dtype=np.int32)


def kernel(numerical_input, categorical_inputs, emb_table,
           bot_0, bot_1, bot_2, bot_3, bot_4, bot_5,
           top_0, top_1, top_2, top_3, top_4,
           top_5, top_6, top_7, top_8, top_9):
    offsets = jnp.arange(NUM_SPARSE, dtype=categorical_inputs.dtype) * VOCAB
    idx = (categorical_inputs + offsets[None, :]).reshape(_NW, _IDXROWS, 128)
    table128 = emb_table.reshape(_PACKED_V, 128)
    pidx = lax.shift_right_logical(idx, 3)
    emb_flat = _sc_gather()(table128, idx, pidx)
    emb2d = emb_flat.reshape(B, NUM_SPARSE * EMB_DIM)

    w0a = top_0[:EMB_DIM]
    w0z = jnp.zeros((N_FEAT * N_FEAT, top_0.shape[1]), top_0.dtype)
    w0z = w0z.at[_PAIR_POS].set(top_0[EMB_DIM:])

    row = lambda b: b.reshape(1, -1)
    bf = lambda w: w.astype(jnp.bfloat16)
    return _tc_dlrm(numerical_input, emb2d,
                    bot_0, row(bot_1), bf(bot_2), row(bot_3), bot_4, row(bot_5),
                    w0a, bf(w0z), row(top_1), bf(top_2), row(top_3), bf(top_4),
                    row(top_5), bf(top_6), row(top_7), top_8, row(top_9))


# speedup vs baseline: 1.0467x; 1.0467x over previous
import functools

import jax
import jax.numpy as jnp
import numpy as np
from jax import lax
from jax.experimental import pallas as pl
from jax.experimental.pallas import tpu as pltpu
from jax.experimental.pallas import tpu_sc as plsc

B = 16384
NUM_SPARSE = 26
VOCAB = 100000
EMB_DIM = 16
N_FEAT = NUM_SPARSE + 1

_NC, _NS = 2, 16
_NW = _NC * _NS
_TOTAL_ROWS = B * NUM_SPARSE
_ROWS_PER_W = _TOTAL_ROWS // _NW
_PACKED_V = 325000
_CHUNK = 256
_NCHUNKS = _ROWS_PER_W // _CHUNK
_IDXROWS = _ROWS_PER_W // 128
_OUT_PACKED = _TOTAL_ROWS // 8
_OPW = _ROWS_PER_W // 8


def _sc_gather_body(table_hbm, idx_hbm, out_hbm, idx_v, rows_v, sem):
    wid = lax.axis_index("s") * _NC + lax.axis_index("c")
    pltpu.sync_copy(idx_hbm.at[wid], idx_v)
    obase = wid * _ROWS_PER_W

    def group(g, carry):
        handles = []
        for j in range(8):
            handles.append(
                pltpu.async_copy(
                    table_hbm.at[idx_v.at[g * 8 + j]],
                    rows_v.at[pl.ds(j * 128, 128)],
                    sem,
                )
            )
        for h in handles:
            h.wait()
        start = pl.multiple_of(obase + g * 1024, 1024)
        pltpu.sync_copy(rows_v, out_hbm.at[pl.ds(start, 1024)])
        return carry

    lax.fori_loop(0, _IDXROWS // 8, group, 0)


@functools.cache
def _sc_gather():
    return pl.kernel(
        _sc_gather_body,
        mesh=plsc.VectorSubcoreMesh(core_axis_name="c", subcore_axis_name="s"),
        out_type=jax.ShapeDtypeStruct((_TOTAL_ROWS, EMB_DIM), jnp.float32),
        scratch_types=[
            pltpu.VMEM((_IDXROWS, 128), jnp.int32),
            pltpu.VMEM((1024, EMB_DIM), jnp.float32),
            pltpu.SemaphoreType.DMA,
        ],
        compiler_params=pltpu.CompilerParams(use_tc_tiling_on_sc=False),
    )


_BB = 512


def _tc_dlrm_body(num_ref, emb_ref, b0w, b0b, b1w, b1b, b2w, b2b,
                  w0a, w0z, t0b, t1w, t1b, t2w, t2b, t3w, t3b, t4w, t4b,
                  out_ref):
    f32 = jnp.float32
    bf16 = jnp.bfloat16
    x = num_ref[...]
    h = jax.nn.relu(jnp.dot(x, b0w[...], preferred_element_type=f32) + b0b[...])
    h = jax.nn.relu(jnp.dot(h.astype(bf16), b1w[...], preferred_element_type=f32)
                    + b1b[...])
    bot = jax.nn.relu(jnp.dot(h, b2w[...], preferred_element_type=f32) + b2b[...])

    emb = emb_ref[...].reshape(_BB, NUM_SPARSE, EMB_DIM)
    xcat = jnp.concatenate([bot.reshape(_BB, 1, EMB_DIM), emb], axis=1)
    z = lax.dot_general(xcat, xcat, (((2,), (2,)), ((0,), (0,))),
                        preferred_element_type=f32)
    zflat = z.reshape(_BB, N_FEAT * N_FEAT)

    t = jnp.dot(bot, w0a[...], preferred_element_type=f32)
    t = t + jnp.dot(zflat.astype(bf16), w0z[...], preferred_element_type=f32)
    t = jax.nn.relu(t + t0b[...])
    t = jax.nn.relu(jnp.dot(t.astype(bf16), t1w[...], preferred_element_type=f32)
                    + t1b[...])
    t = jax.nn.relu(jnp.dot(t.astype(bf16), t2w[...], preferred_element_type=f32)
                    + t2b[...])
    t = jax.nn.relu(jnp.dot(t.astype(bf16), t3w[...], preferred_element_type=f32)
                    + t3b[...])
    out_ref[...] = jnp.dot(t, t4w[...], preferred_element_type=f32) + t4b[...]


def _tc_dlrm(num, emb2d, b0w, b0b, b1w, b1b, b2w, b2b,
             w0a, w0z, t0b, t1w, t1b, t2w, t2b, t3w, t3b, t4w, t4b):
    grid = (B // _BB,)
    full = lambda a: pl.BlockSpec(a.shape, lambda i: (0,) * a.ndim)
    in_specs = [
        pl.BlockSpec((_BB, num.shape[1]), lambda i: (i, 0)),
        pl.BlockSpec((_BB * NUM_SPARSE, EMB_DIM), lambda i: (i, 0)),
    ] + [full(a) for a in (b0w, b0b, b1w, b1b, b2w, b2b,
                           w0a, w0z, t0b, t1w, t1b, t2w, t2b, t3w, t3b,
                           t4w, t4b)]
    return pl.pallas_call(
        _tc_dlrm_body,
        grid=grid,
        in_specs=in_specs,
        out_specs=pl.BlockSpec((_BB, 1), lambda i: (i, 0)),
        out_shape=jax.ShapeDtypeStruct((B, 1), jnp.float32),
    )(num, emb2d, b0w, b0b, b1w, b1b, b2w, b2b,
      w0a, w0z, t0b, t1w, t1b, t2w, t2b, t3w, t3b, t4w, t4b)


_LI, _LJ = np.tril_indices(N_FEAT, -1)
_PAIR_POS = np.asarray(_LI * N_FEAT + _LJ, dtype=np.int32)


def kernel(numerical_input, categorical_inputs, emb_table,
           bot_0, bot_1, bot_2, bot_3, bot_4, bot_5,
           top_0, top_1, top_2, top_3, top_4,
           top_5, top_6, top_7, top_8, top_9):
    offsets = jnp.arange(NUM_SPARSE, dtype=categorical_inputs.dtype) * VOCAB
    idx = (categorical_inputs + offsets[None, :]).reshape(_NW, _IDXROWS, 128)
    emb_flat = _sc_gather()(emb_table, idx)

    w0a = top_0[:EMB_DIM]
    w0z = jnp.zeros((N_FEAT * N_FEAT, top_0.shape[1]), top_0.dtype)
    w0z = w0z.at[_PAIR_POS].set(top_0[EMB_DIM:])

    row = lambda b: b.reshape(1, -1)
    bf = lambda w: w.astype(jnp.bfloat16)
    return _tc_dlrm(numerical_input, emb_flat,
                    bot_0, row(bot_1), bf(bot_2), row(bot_3), bot_4, row(bot_5),
                    w0a, bf(w0z), row(top_1), bf(top_2), row(top_3), bf(top_4),
                    row(top_5), bf(top_6), row(top_7), top_8, row(top_9))


# speedup vs baseline: 1.6996x; 1.6237x over previous
import functools

import jax
import jax.numpy as jnp
import numpy as np
from jax import lax
from jax.experimental import pallas as pl
from jax.experimental.pallas import tpu as pltpu
from jax.experimental.pallas import tpu_sc as plsc

B = 16384
NUM_SPARSE = 26
VOCAB = 100000
EMB_DIM = 16
N_FEAT = NUM_SPARSE + 1

_NC, _NS = 2, 16
_NW = _NC * _NS
_TOTAL_ROWS = B * NUM_SPARSE
_ROWS_PER_W = _TOTAL_ROWS // _NW
_PACKED_V = 325000
_CHUNK = 256
_NCHUNKS = _ROWS_PER_W // _CHUNK
_IDXROWS = _ROWS_PER_W // 128
_OUT_PACKED = _TOTAL_ROWS // 8
_OPW = _ROWS_PER_W // 8


_SCOLS = 1536
_SLAB = _SCOLS // 8
_NSLABS = (NUM_SPARSE * VOCAB) // _SCOLS
_TCOLS = 1024
_SPW = _NSLABS // _NW
_SREM = _NSLABS - _SPW * _NW


def _sc_pack_body(tT_hbm, tail_hbm, out_hbm, slab_v, pbuf, sem):
    i32 = jnp.int32
    wid = lax.axis_index("s") * _NC + lax.axis_index("c")
    iota16 = lax.iota(i32, 16)
    lane_hi = lax.shift_right_logical(iota16, 3)
    cbase = lax.shift_left(lax.bitwise_and(iota16, 7), 4)
    start = wid * _SPW + jnp.minimum(wid, _SREM)
    count = jnp.where(wid < _SREM, _SPW + 1, _SPW)

    def shuffle(ngrp):
        def grp_body(grp, carry):
            rowv = 2 * grp + lane_hi
            for d in range(EMB_DIM):
                v = slab_v[d, pl.ds(grp * 16, 16)]
                plsc.store_scatter(pbuf, [rowv, cbase + d], v)
            return carry
        lax.fori_loop(0, ngrp, grp_body, 0)

    def slab(s, carry):
        g = start + s
        pltpu.sync_copy(tT_hbm.at[:, pl.ds(g * _SCOLS, _SCOLS)], slab_v)
        shuffle(_SCOLS // 16)
        pltpu.sync_copy(pbuf, out_hbm.at[pl.ds(g * _SLAB, _SLAB)])
        return carry

    lax.fori_loop(0, count, slab, 0)

    @pl.when(wid == _NW - 1)
    def _tail():
        pltpu.sync_copy(
            tT_hbm.at[:, pl.ds(_NSLABS * _SCOLS, _TCOLS)],
            slab_v.at[:, pl.ds(0, _TCOLS)],
        )
        shuffle(_TCOLS // 16)
        pltpu.sync_copy(
            pbuf.at[pl.ds(0, _TCOLS // 8)],
            out_hbm.at[pl.ds(_NSLABS * _SLAB, _TCOLS // 8)],
        )
        pltpu.sync_copy(tail_hbm, pbuf.at[pl.ds(0, 8)])
        pltpu.sync_copy(
            pbuf.at[pl.ds(0, 8)],
            out_hbm.at[pl.ds(_NSLABS * _SLAB + _TCOLS // 8, 8)],
        )


@functools.cache
def _sc_pack():
    return pl.kernel(
        _sc_pack_body,
        mesh=plsc.VectorSubcoreMesh(core_axis_name="c", subcore_axis_name="s"),
        out_type=jax.ShapeDtypeStruct((_PACKED_V, 128), jnp.float32),
        scratch_types=[
            pltpu.VMEM((EMB_DIM, _SCOLS), jnp.float32),
            pltpu.VMEM((_SLAB, 128), jnp.float32),
            pltpu.SemaphoreType.DMA,
        ],
        compiler_params=pltpu.CompilerParams(
            use_tc_tiling_on_sc=True, needs_layout_passes=False),
    )


def _sc_gather_body(table_hbm, idx_hbm, pidx_hbm, out_hbm,
                    idx_v, pidx_v, gbuf, obuf, sem):
    i32 = jnp.int32
    wid = lax.axis_index("s") * _NC + lax.axis_index("c")
    pltpu.sync_copy(idx_hbm.at[wid], idx_v)
    pltpu.sync_copy(pidx_hbm.at[wid], pidx_v)

    iota16 = lax.iota(i32, 16)
    lane_hi = lax.shift_right_logical(iota16, 3)
    cbase = lax.shift_left(lax.bitwise_and(iota16, 7), 4)
    obase = wid * _OPW

    def chunk(c, carry):
        hs = [
            pltpu.async_copy(
                table_hbm.at[pidx_v.at[2 * c + h]],
                gbuf.at[pl.ds(h * 128, 128)],
                sem,
            )
            for h in range(2)
        ]
        for h in hs:
            h.wait()
        for g in range(16):
            ov = lax.shift_left(
                lax.bitwise_and(idx_v[2 * c + g // 8, pl.ds((g % 8) * 16, 16)], 7),
                4)
            growvec = g * 16 + iota16
            orowvec = (lax.rem(c, 4) * 32 + 2 * g) + lane_hi
            for i in range(16):
                val = plsc.load_gather(gbuf, [growvec, ov + i])
                plsc.store_scatter(obuf, [orowvec, cbase + i], val)

        @pl.when(lax.rem(c, 4) == 3)
        def _flush():
            start = pl.multiple_of(obase + (c // 4) * 128, 128)
            pltpu.sync_copy(obuf, out_hbm.at[pl.ds(start, 128)])

        return carry

    lax.fori_loop(0, _NCHUNKS, chunk, 0)


@functools.cache
def _sc_gather():
    return pl.kernel(
        _sc_gather_body,
        mesh=plsc.VectorSubcoreMesh(core_axis_name="c", subcore_axis_name="s"),
        out_type=jax.ShapeDtypeStruct((_OUT_PACKED, 128), jnp.float32),
        scratch_types=[
            pltpu.VMEM((_IDXROWS, 128), jnp.int32),
            pltpu.VMEM((_IDXROWS, 128), jnp.int32),
            pltpu.VMEM((_CHUNK, 128), jnp.float32),
            pltpu.VMEM((128, 128), jnp.float32),
            pltpu.SemaphoreType.DMA,
        ],
        compiler_params=pltpu.CompilerParams(
            use_tc_tiling_on_sc=True, needs_layout_passes=False),
    )


_BB = 512


def _tc_dlrm_body(num_ref, emb_ref, b0w, b0b, b1w, b1b, b2w, b2b,
                  w0a, w0z, t0b, t1w, t1b, t2w, t2b, t3w, t3b, t4w, t4b,
                  out_ref):
    f32 = jnp.float32
    bf16 = jnp.bfloat16
    x = num_ref[...]
    h = jax.nn.relu(jnp.dot(x, b0w[...], preferred_element_type=f32) + b0b[...])
    h = jax.nn.relu(jnp.dot(h.astype(bf16), b1w[...], preferred_element_type=f32)
                    + b1b[...])
    bot = jax.nn.relu(jnp.dot(h, b2w[...], preferred_element_type=f32) + b2b[...])

    emb = emb_ref[...].reshape(_BB, NUM_SPARSE, EMB_DIM)
    xcat = jnp.concatenate([bot.reshape(_BB, 1, EMB_DIM), emb], axis=1)
    z = lax.dot_general(xcat, xcat, (((2,), (2,)), ((0,), (0,))),
                        preferred_element_type=f32)
    zflat = z.reshape(_BB, N_FEAT * N_FEAT)

    t = jnp.dot(bot, w0a[...], preferred_element_type=f32)
    t = t + jnp.dot(zflat.astype(bf16), w0z[...], preferred_element_type=f32)
    t = jax.nn.relu(t + t0b[...])
    t = jax.nn.relu(jnp.dot(t.astype(bf16), t1w[...], preferred_element_type=f32)
                    + t1b[...])
    t = jax.nn.relu(jnp.dot(t.astype(bf16), t2w[...], preferred_element_type=f32)
                    + t2b[...])
    t = jax.nn.relu(jnp.dot(t.astype(bf16), t3w[...], preferred_element_type=f32)
                    + t3b[...])
    out_ref[...] = jnp.dot(t, t4w[...], preferred_element_type=f32) + t4b[...]


def _tc_dlrm(num, emb2d, b0w, b0b, b1w, b1b, b2w, b2b,
             w0a, w0z, t0b, t1w, t1b, t2w, t2b, t3w, t3b, t4w, t4b):
    grid = (B // _BB,)
    full = lambda a: pl.BlockSpec(a.shape, lambda i: (0,) * a.ndim)
    in_specs = [
        pl.BlockSpec((_BB, num.shape[1]), lambda i: (i, 0)),
        pl.BlockSpec((_BB, emb2d.shape[1]), lambda i: (i, 0)),
    ] + [full(a) for a in (b0w, b0b, b1w, b1b, b2w, b2b,
                           w0a, w0z, t0b, t1w, t1b, t2w, t2b, t3w, t3b,
                           t4w, t4b)]
    return pl.pallas_call(
        _tc_dlrm_body,
        grid=grid,
        in_specs=in_specs,
        out_specs=pl.BlockSpec((_BB, 1), lambda i: (i, 0)),
        out_shape=jax.ShapeDtypeStruct((B, 1), jnp.float32),
    )(num, emb2d, b0w, b0b, b1w, b1b, b2w, b2b,
      w0a, w0z, t0b, t1w, t1b, t2w, t2b, t3w, t3b, t4w, t4b)


_LI, _LJ = np.tril_indices(N_FEAT, -1)
_PAIR_POS = np.asarray(_LI * N_FEAT + _LJ, dtype=np.int32)


def kernel(numerical_input, categorical_inputs, emb_table,
           bot_0, bot_1, bot_2, bot_3, bot_4, bot_5,
           top_0, top_1, top_2, top_3, top_4,
           top_5, top_6, top_7, top_8, top_9):
    offsets = jnp.arange(NUM_SPARSE, dtype=categorical_inputs.dtype) * VOCAB
    idx = (categorical_inputs + offsets[None, :]).reshape(_NW, _IDXROWS, 128)
    pidx = lax.shift_right_logical(idx, 3)
    tail_pack = emb_table[_PACKED_V * 8 - 64:].reshape(8, 128)
    table128 = _sc_pack()(emb_table.T, tail_pack)
    emb_flat = _sc_gather()(table128, idx, pidx)
    emb2d = emb_flat.reshape(B, NUM_SPARSE * EMB_DIM)

    w0a = top_0[:EMB_DIM]
    w0z = jnp.zeros((N_FEAT * N_FEAT, top_0.shape[1]), top_0.dtype)
    w0z = w0z.at[_PAIR_POS].set(top_0[EMB_DIM:])

    row = lambda b: b.reshape(1, -1)
    bf = lambda w: w.astype(jnp.bfloat16)
    return _tc_dlrm(numerical_input, emb2d,
                    bot_0, row(bot_1), bf(bot_2), row(bot_3), bot_4, row(bot_5),
                    w0a, bf(w0z), row(top_1), bf(top_2), row(top_3), bf(top_4),
                    row(top_5), bf(top_6), row(top_7), top_8, row(top_9))


# speedup vs baseline: 2.1281x; 1.2521x over previous
import functools

import jax
import jax.numpy as jnp
import numpy as np
from jax import lax
from jax.experimental import pallas as pl
from jax.experimental.pallas import tpu as pltpu
from jax.experimental.pallas import tpu_sc as plsc

B = 16384
NUM_SPARSE = 26
VOCAB = 100000
EMB_DIM = 16
N_FEAT = NUM_SPARSE + 1

_NC, _NS = 2, 16
_NW = _NC * _NS
_TOTAL_ROWS = B * NUM_SPARSE
_ROWS_PER_W = _TOTAL_ROWS // _NW
_PACKED_V = 325000
_CHUNK = 256
_NCHUNKS = _ROWS_PER_W // _CHUNK
_IDXROWS = _ROWS_PER_W // 128
_OUT_PACKED = _TOTAL_ROWS // 8
_OPW = _ROWS_PER_W // 8


_SCOLS = 1536
_SLAB = _SCOLS // 8
_NSLABS = (NUM_SPARSE * VOCAB) // _SCOLS
_TCOLS = 1024
_SPW = _NSLABS // _NW
_SREM = _NSLABS - _SPW * _NW


def _sc_pack_body(tT_hbm, tail_hbm, out_hbm, slab_v, pbuf, sem):
    i32 = jnp.int32
    wid = lax.axis_index("s") * _NC + lax.axis_index("c")
    iota16 = lax.iota(i32, 16)
    lane_hi = lax.shift_right_logical(iota16, 3)
    cbase = lax.shift_left(lax.bitwise_and(iota16, 7), 4)
    start = wid * _SPW + jnp.minimum(wid, _SREM)
    count = jnp.where(wid < _SREM, _SPW + 1, _SPW)

    def shuffle(buf, ngrp):
        def grp_body(grp, carry):
            rowv = 2 * grp + lane_hi
            for d in range(EMB_DIM):
                v = buf[d, pl.ds(grp * 16, 16)]
                plsc.store_scatter(pbuf, [rowv, cbase + d], v)
            return carry
        lax.fori_loop(0, ngrp, grp_body, 0)

    pltpu.async_copy(
        tT_hbm.at[:, pl.ds(start * _SCOLS, _SCOLS)], slab_v.at[0], sem)

    def slab(s, carry):
        cur = lax.rem(s, 2)
        pltpu.make_async_copy(
            tT_hbm.at[:, pl.ds(0, _SCOLS)], slab_v.at[cur], sem).wait()

        @pl.when(s + 1 < count)
        def _prefetch():
            pltpu.async_copy(
                tT_hbm.at[:, pl.ds((start + s + 1) * _SCOLS, _SCOLS)],
                slab_v.at[1 - cur], sem)

        shuffle(slab_v.at[cur], _SCOLS // 16)
        pltpu.sync_copy(pbuf, out_hbm.at[pl.ds((start + s) * _SLAB, _SLAB)])
        return carry

    lax.fori_loop(0, count, slab, 0)

    @pl.when(wid == _NW - 1)
    def _tail():
        pltpu.sync_copy(
            tT_hbm.at[:, pl.ds(_NSLABS * _SCOLS, _TCOLS)],
            slab_v.at[0].at[:, pl.ds(0, _TCOLS)],
        )
        shuffle(slab_v.at[0], _TCOLS // 16)
        pltpu.sync_copy(
            pbuf.at[pl.ds(0, _TCOLS // 8)],
            out_hbm.at[pl.ds(_NSLABS * _SLAB, _TCOLS // 8)],
        )
        pltpu.sync_copy(tail_hbm, pbuf.at[pl.ds(0, 8)])
        pltpu.sync_copy(
            pbuf.at[pl.ds(0, 8)],
            out_hbm.at[pl.ds(_NSLABS * _SLAB + _TCOLS // 8, 8)],
        )


@functools.cache
def _sc_pack():
    return pl.kernel(
        _sc_pack_body,
        mesh=plsc.VectorSubcoreMesh(core_axis_name="c", subcore_axis_name="s"),
        out_type=jax.ShapeDtypeStruct((_PACKED_V, 128), jnp.float32),
        scratch_types=[
            pltpu.VMEM((2, EMB_DIM, _SCOLS), jnp.float32),
            pltpu.VMEM((_SLAB, 128), jnp.float32),
            pltpu.SemaphoreType.DMA,
        ],
        compiler_params=pltpu.CompilerParams(
            use_tc_tiling_on_sc=True, needs_layout_passes=False),
    )


def _sc_gather_body(table_hbm, idx_hbm, pidx_hbm, out_hbm,
                    idx_v, pidx_v, gbuf, obuf, sem):
    i32 = jnp.int32
    wid = lax.axis_index("s") * _NC + lax.axis_index("c")
    pltpu.sync_copy(idx_hbm.at[wid], idx_v)
    pltpu.sync_copy(pidx_hbm.at[wid], pidx_v)

    iota16 = lax.iota(i32, 16)
    lane_hi = lax.shift_right_logical(iota16, 3)
    cbase = lax.shift_left(lax.bitwise_and(iota16, 7), 4)
    obase = wid * _OPW

    def fire(c, buf):
        for h in range(2):
            pltpu.async_copy(
                table_hbm.at[pidx_v.at[2 * c + h]],
                gbuf.at[buf].at[pl.ds(h * 128, 128)],
                sem,
            )

    fire(0, 0)

    def chunk(c, carry):
        cur = lax.rem(c, 2)
        for _ in range(2):
            pltpu.make_async_copy(
                table_hbm.at[pidx_v.at[0]], gbuf.at[0].at[pl.ds(0, 128)], sem
            ).wait()

        @pl.when(c + 1 < _NCHUNKS)
        def _prefetch():
            fire(c + 1, 1 - cur)

        gcur = gbuf.at[cur]
        for g in range(16):
            ov = lax.shift_left(
                lax.bitwise_and(idx_v[2 * c + g // 8, pl.ds((g % 8) * 16, 16)], 7),
                4)
            growvec = g * 16 + iota16
            orowvec = (lax.rem(c, 4) * 32 + 2 * g) + lane_hi
            for i in range(16):
                val = plsc.load_gather(gcur, [growvec, ov + i])
                plsc.store_scatter(obuf, [orowvec, cbase + i], val)

        @pl.when(lax.rem(c, 4) == 3)
        def _flush():
            start = pl.multiple_of(obase + (c // 4) * 128, 128)
            pltpu.sync_copy(obuf, out_hbm.at[pl.ds(start, 128)])

        return carry

    lax.fori_loop(0, _NCHUNKS, chunk, 0)


@functools.cache
def _sc_gather():
    return pl.kernel(
        _sc_gather_body,
        mesh=plsc.VectorSubcoreMesh(core_axis_name="c", subcore_axis_name="s"),
        out_type=jax.ShapeDtypeStruct((_OUT_PACKED, 128), jnp.float32),
        scratch_types=[
            pltpu.VMEM((_IDXROWS, 128), jnp.int32),
            pltpu.VMEM((_IDXROWS, 128), jnp.int32),
            pltpu.VMEM((2, _CHUNK, 128), jnp.float32),
            pltpu.VMEM((128, 128), jnp.float32),
            pltpu.SemaphoreType.DMA,
        ],
        compiler_params=pltpu.CompilerParams(
            use_tc_tiling_on_sc=True, needs_layout_passes=False),
    )


_BB = 512


def _tc_dlrm_body(num_ref, emb_ref, b0w, b0b, b1w, b1b, b2w, b2b,
                  w0a, w0z, t0b, t1w, t1b, t2w, t2b, t3w, t3b, t4w, t4b,
                  out_ref):
    f32 = jnp.float32
    bf16 = jnp.bfloat16
    x = num_ref[...]
    h = jax.nn.relu(jnp.dot(x, b0w[...], preferred_element_type=f32) + b0b[...])
    h = jax.nn.relu(jnp.dot(h.astype(bf16), b1w[...], preferred_element_type=f32)
                    + b1b[...])
    bot = jax.nn.relu(jnp.dot(h, b2w[...], preferred_element_type=f32) + b2b[...])

    emb = emb_ref[...].reshape(_BB, NUM_SPARSE, EMB_DIM)
    xcat = jnp.concatenate([bot.reshape(_BB, 1, EMB_DIM), emb], axis=1)
    z = lax.dot_general(xcat, xcat, (((2,), (2,)), ((0,), (0,))),
                        preferred_element_type=f32)
    zflat = z.reshape(_BB, N_FEAT * N_FEAT)

    t = jnp.dot(bot, w0a[...], preferred_element_type=f32)
    t = t + jnp.dot(zflat.astype(bf16), w0z[...], preferred_element_type=f32)
    t = jax.nn.relu(t + t0b[...])
    t = jax.nn.relu(jnp.dot(t.astype(bf16), t1w[...], preferred_element_type=f32)
                    + t1b[...])
    t = jax.nn.relu(jnp.dot(t.astype(bf16), t2w[...], preferred_element_type=f32)
                    + t2b[...])
    t = jax.nn.relu(jnp.dot(t.astype(bf16), t3w[...], preferred_element_type=f32)
                    + t3b[...])
    out_ref[...] = jnp.dot(t, t4w[...], preferred_element_type=f32) + t4b[...]


def _tc_dlrm(num, emb2d, b0w, b0b, b1w, b1b, b2w, b2b,
             w0a, w0z, t0b, t1w, t1b, t2w, t2b, t3w, t3b, t4w, t4b):
    grid = (B // _BB,)
    full = lambda a: pl.BlockSpec(a.shape, lambda i: (0,) * a.ndim)
    in_specs = [
        pl.BlockSpec((_BB, num.shape[1]), lambda i: (i, 0)),
        pl.BlockSpec((_BB, emb2d.shape[1]), lambda i: (i, 0)),
    ] + [full(a) for a in (b0w, b0b, b1w, b1b, b2w, b2b,
                           w0a, w0z, t0b, t1w, t1b, t2w, t2b, t3w, t3b,
                           t4w, t4b)]
    return pl.pallas_call(
        _tc_dlrm_body,
        grid=grid,
        in_specs=in_specs,
        out_specs=pl.BlockSpec((_BB, 1), lambda i: (i, 0)),
        out_shape=jax.ShapeDtypeStruct((B, 1), jnp.float32),
    )(num, emb2d, b0w, b0b, b1w, b1b, b2w, b2b,
      w0a, w0z, t0b, t1w, t1b, t2w, t2b, t3w, t3b, t4w, t4b)


_LI, _LJ = np.tril_indices(N_FEAT, -1)
_PAIR_POS = np.asarray(_LI * N_FEAT + _LJ, dtype=np.int32)


def kernel(numerical_input, categorical_inputs, emb_table,
           bot_0, bot_1, bot_2, bot_3, bot_4, bot_5,
           top_0, top_1, top_2, top_3, top_4,
           top_5, top_6, top_7, top_8, top_9):
    offsets = jnp.arange(NUM_SPARSE, dtype=categorical_inputs.dtype) * VOCAB
    idx = (categorical_inputs + offsets[None, :]).reshape(_NW, _IDXROWS, 128)
    pidx = lax.shift_right_logical(idx, 3)
    tail_pack = emb_table[_PACKED_V * 8 - 64:].reshape(8, 128)
    table128 = _sc_pack()(emb_table.T, tail_pack)
    emb_flat = _sc_gather()(table128, idx, pidx)
    emb2d = emb_flat.reshape(B, NUM_SPARSE * EMB_DIM)

    w0a = top_0[:EMB_DIM]
    w0z = jnp.zeros((N_FEAT * N_FEAT, top_0.shape[1]), top_0.dtype)
    w0z = w0z.at[_PAIR_POS].set(top_0[EMB_DIM:])

    row = lambda b: b.reshape(1, -1)
    bf = lambda w: w.astype(jnp.bfloat16)
    return _tc_dlrm(numerical_input, emb2d,
                    bot_0, row(bot_1), bf(bot_2), row(bot_3), bot_4, row(bot_5),
                    w0a, bf(w0z), row(top_1), bf(top_2), row(top_3), bf(top_4),
                    row(top_5), bf(top_6), row(top_7), top_8, row(top_9))


# speedup vs baseline: 2.1533x; 1.0119x over previous
import functools

import jax
import jax.numpy as jnp
import numpy as np
from jax import lax
from jax.experimental import pallas as pl
from jax.experimental.pallas import tpu as pltpu
from jax.experimental.pallas import tpu_sc as plsc

B = 16384
NUM_SPARSE = 26
VOCAB = 100000
EMB_DIM = 16
N_FEAT = NUM_SPARSE + 1

_NC, _NS = 2, 16
_NW = _NC * _NS
_TOTAL_ROWS = B * NUM_SPARSE
_ROWS_PER_W = _TOTAL_ROWS // _NW
_PACKED_V = 325000
_CHUNK = 256
_NCHUNKS = _ROWS_PER_W // _CHUNK
_IDXROWS = _ROWS_PER_W // 128
_OUT_PACKED = _TOTAL_ROWS // 8
_OPW = _ROWS_PER_W // 8


_SCOLS = 1536
_SLAB = _SCOLS // 8
_NSLABS = (NUM_SPARSE * VOCAB) // _SCOLS
_TCOLS = 1024
_SPW = _NSLABS // _NW
_SREM = _NSLABS - _SPW * _NW


def _sc_pack_body(tT_hbm, tail_hbm, out_hbm, slab_v, pbuf, sem):
    i32 = jnp.int32
    wid = lax.axis_index("s") * _NC + lax.axis_index("c")
    iota16 = lax.iota(i32, 16)
    lane_hi = lax.shift_right_logical(iota16, 3)
    cbase = lax.shift_left(lax.bitwise_and(iota16, 7), 4)
    start = wid * _SPW + jnp.minimum(wid, _SREM)
    count = jnp.where(wid < _SREM, _SPW + 1, _SPW)

    def shuffle(buf, ngrp):
        def grp_body(grp, carry):
            rowv = 2 * grp + lane_hi
            for d in range(EMB_DIM):
                v = buf[d, pl.ds(grp * 16, 16)]
                plsc.store_scatter(pbuf, [rowv, cbase + d], v)
            return carry
        lax.fori_loop(0, ngrp, grp_body, 0)

    pltpu.async_copy(
        tT_hbm.at[:, pl.ds(start * _SCOLS, _SCOLS)], slab_v.at[0], sem)

    def slab(s, carry):
        cur = lax.rem(s, 2)
        pltpu.make_async_copy(
            tT_hbm.at[:, pl.ds(0, _SCOLS)], slab_v.at[cur], sem).wait()

        @pl.when(s + 1 < count)
        def _prefetch():
            pltpu.async_copy(
                tT_hbm.at[:, pl.ds((start + s + 1) * _SCOLS, _SCOLS)],
                slab_v.at[1 - cur], sem)

        shuffle(slab_v.at[cur], _SCOLS // 16)
        pltpu.sync_copy(pbuf, out_hbm.at[pl.ds((start + s) * _SLAB, _SLAB)])
        return carry

    lax.fori_loop(0, count, slab, 0)

    @pl.when(wid == _NW - 1)
    def _tail():
        pltpu.sync_copy(
            tT_hbm.at[:, pl.ds(_NSLABS * _SCOLS, _TCOLS)],
            slab_v.at[0].at[:, pl.ds(0, _TCOLS)],
        )
        shuffle(slab_v.at[0], _TCOLS // 16)
        pltpu.sync_copy(
            pbuf.at[pl.ds(0, _TCOLS // 8)],
            out_hbm.at[pl.ds(_NSLABS * _SLAB, _TCOLS // 8)],
        )
        pltpu.sync_copy(tail_hbm, pbuf.at[pl.ds(0, 8)])
        pltpu.sync_copy(
            pbuf.at[pl.ds(0, 8)],
            out_hbm.at[pl.ds(_NSLABS * _SLAB + _TCOLS // 8, 8)],
        )


@functools.cache
def _sc_pack():
    return pl.kernel(
        _sc_pack_body,
        mesh=plsc.VectorSubcoreMesh(core_axis_name="c", subcore_axis_name="s"),
        out_type=jax.ShapeDtypeStruct((_PACKED_V, 128), jnp.float32),
        scratch_types=[
            pltpu.VMEM((2, EMB_DIM, _SCOLS), jnp.float32),
            pltpu.VMEM((_SLAB, 128), jnp.float32),
            pltpu.SemaphoreType.DMA,
        ],
        compiler_params=pltpu.CompilerParams(
            use_tc_tiling_on_sc=True, needs_layout_passes=False),
    )


def _sc_gather_body(table_hbm, idx_hbm, pidx_hbm, out_hbm,
                    idx_v, pidx_v, gbuf, obuf, sem):
    i32 = jnp.int32
    wid = lax.axis_index("s") * _NC + lax.axis_index("c")
    pltpu.sync_copy(idx_hbm.at[wid], idx_v)
    pltpu.sync_copy(pidx_hbm.at[wid], pidx_v)

    iota16 = lax.iota(i32, 16)
    lane_hi = lax.shift_right_logical(iota16, 3)
    cbase = lax.shift_left(lax.bitwise_and(iota16, 7), 4)
    obase = wid * _OPW

    def fire(c, buf):
        for h in range(2):
            pltpu.async_copy(
                table_hbm.at[pidx_v.at[2 * c + h]],
                gbuf.at[buf].at[pl.ds(h * 128, 128)],
                sem,
            )

    fire(0, 0)

    def chunk(c, carry):
        cur = lax.rem(c, 2)
        for _ in range(2):
            pltpu.make_async_copy(
                table_hbm.at[pidx_v.at[0]], gbuf.at[0].at[pl.ds(0, 128)], sem
            ).wait()

        @pl.when(c + 1 < _NCHUNKS)
        def _prefetch():
            fire(c + 1, 1 - cur)

        gcur = gbuf.at[cur]
        for g in range(16):
            ov = lax.shift_left(
                lax.bitwise_and(idx_v[2 * c + g // 8, pl.ds((g % 8) * 16, 16)], 7),
                4)
            growvec = g * 16 + iota16
            orowvec = (lax.rem(c, 4) * 32 + 2 * g) + lane_hi
            for i in range(16):
                val = plsc.load_gather(gcur, [growvec, ov + i])
                plsc.store_scatter(obuf, [orowvec, cbase + i], val)

        @pl.when(lax.rem(c, 4) == 3)
        def _flush():
            start = pl.multiple_of(obase + (c // 4) * 128, 128)
            pltpu.sync_copy(obuf, out_hbm.at[pl.ds(start, 128)])

        return carry

    lax.fori_loop(0, _NCHUNKS, chunk, 0)


@functools.cache
def _sc_gather():
    return pl.kernel(
        _sc_gather_body,
        mesh=plsc.VectorSubcoreMesh(core_axis_name="c", subcore_axis_name="s"),
        out_type=jax.ShapeDtypeStruct((_OUT_PACKED, 128), jnp.float32),
        scratch_types=[
            pltpu.VMEM((_IDXROWS, 128), jnp.int32),
            pltpu.VMEM((_IDXROWS, 128), jnp.int32),
            pltpu.VMEM((2, _CHUNK, 128), jnp.float32),
            pltpu.VMEM((128, 128), jnp.float32),
            pltpu.SemaphoreType.DMA,
        ],
        compiler_params=pltpu.CompilerParams(
            use_tc_tiling_on_sc=True, needs_layout_passes=False),
    )


_BB = 1024


def _tc_dlrm_body(num_ref, emb_ref, b0w, b0b, b1w, b1b, b2w, b2b,
                  w0a, w0z, t0b, t1w, t1b, t2w, t2b, t3w, t3b, t4w, t4b,
                  out_ref):
    f32 = jnp.float32
    bf16 = jnp.bfloat16
    x = num_ref[...]
    h = jax.nn.relu(jnp.dot(x, b0w[...], preferred_element_type=f32) + b0b[...])
    h = jax.nn.relu(jnp.dot(h.astype(bf16), b1w[...], preferred_element_type=f32)
                    + b1b[...])
    bot = jax.nn.relu(jnp.dot(h, b2w[...], preferred_element_type=f32) + b2b[...])

    emb = emb_ref[...].reshape(_BB, NUM_SPARSE, EMB_DIM)
    xcat = jnp.concatenate([bot.reshape(_BB, 1, EMB_DIM), emb], axis=1)
    z = lax.dot_general(xcat, xcat, (((2,), (2,)), ((0,), (0,))),
                        preferred_element_type=f32)
    zflat = z.reshape(_BB, N_FEAT * N_FEAT)

    t = jnp.dot(bot, w0a[...], preferred_element_type=f32)
    t = t + jnp.dot(zflat.astype(bf16), w0z[...], preferred_element_type=f32)
    t = jax.nn.relu(t + t0b[...])
    t = jax.nn.relu(jnp.dot(t.astype(bf16), t1w[...], preferred_element_type=f32)
                    + t1b[...])
    t = jax.nn.relu(jnp.dot(t.astype(bf16), t2w[...], preferred_element_type=f32)
                    + t2b[...])
    t = jax.nn.relu(jnp.dot(t.astype(bf16), t3w[...], preferred_element_type=f32)
                    + t3b[...])
    out_ref[...] = jnp.dot(t, t4w[...], preferred_element_type=f32) + t4b[...]


def _tc_dlrm(num, emb2d, b0w, b0b, b1w, b1b, b2w, b2b,
             w0a, w0z, t0b, t1w, t1b, t2w, t2b, t3w, t3b, t4w, t4b):
    grid = (B // _BB,)
    full = lambda a: pl.BlockSpec(a.shape, lambda i: (0,) * a.ndim)
    in_specs = [
        pl.BlockSpec((_BB, num.shape[1]), lambda i: (i, 0)),
        pl.BlockSpec((_BB, emb2d.shape[1]), lambda i: (i, 0)),
    ] + [full(a) for a in (b0w, b0b, b1w, b1b, b2w, b2b,
                           w0a, w0z, t0b, t1w, t1b, t2w, t2b, t3w, t3b,
                           t4w, t4b)]
    return pl.pallas_call(
        _tc_dlrm_body,
        grid=grid,
        in_specs=in_specs,
        out_specs=pl.BlockSpec((_BB, 1), lambda i: (i, 0)),
        out_shape=jax.ShapeDtypeStruct((B, 1), jnp.float32),
    )(num, emb2d, b0w, b0b, b1w, b1b, b2w, b2b,
      w0a, w0z, t0b, t1w, t1b, t2w, t2b, t3w, t3b, t4w, t4b)


_LI, _LJ = np.tril_indices(N_FEAT, -1)
_PAIR_POS = np.asarray(_LI * N_FEAT + _LJ, dtype=np.int32)


def kernel(numerical_input, categorical_inputs, emb_table,
           bot_0, bot_1, bot_2, bot_3, bot_4, bot_5,
           top_0, top_1, top_2, top_3, top_4,
           top_5, top_6, top_7, top_8, top_9):
    offsets = jnp.arange(NUM_SPARSE, dtype=categorical_inputs.dtype) * VOCAB
    idx = (categorical_inputs + offsets[None, :]).reshape(_NW, _IDXROWS, 128)
    pidx = lax.shift_right_logical(idx, 3)
    tail_pack = emb_table[_PACKED_V * 8 - 64:].reshape(8, 128)
    table128 = _sc_pack()(emb_table.T, tail_pack)
    emb_flat = _sc_gather()(table128, idx, pidx)
    emb2d = emb_flat.reshape(B, NUM_SPARSE * EMB_DIM)

    w0a = top_0[:EMB_DIM]
    w0z = jnp.zeros((N_FEAT * N_FEAT, top_0.shape[1]), top_0.dtype)
    w0z = w0z.at[_PAIR_POS].set(top_0[EMB_DIM:])

    row = lambda b: b.reshape(1, -1)
    bf = lambda w: w.astype(jnp.bfloat16)
    return _tc_dlrm(numerical_input, emb2d,
                    bot_0, row(bot_1), bf(bot_2), row(bot_3), bot_4, row(bot_5),
                    w0a, bf(w0z), row(top_1), bf(top_2), row(top_3), bf(top_4),
                    row(top_5), bf(top_6), row(top_7), top_8, row(top_9))
